# baseline (speedup 1.0000x reference)
import jax
import jax.numpy as jnp
from jax.experimental import pallas as pl

NUM_CLASSES = 1000
_BB = 16


def _body(x_ref, o_ref):
    xi = x_ref[...]
    iota = jax.lax.broadcasted_iota(jnp.int32, o_ref.shape, 2)
    o_ref[...] = (iota == xi).astype(jnp.float32)


def kernel(x):
    B, F, _ = x.shape
    return pl.pallas_call(
        _body,
        grid=(B // _BB,),
        in_specs=[pl.BlockSpec((_BB, F, 1), lambda i: (i, 0, 0))],
        out_specs=pl.BlockSpec((_BB, F, NUM_CLASSES), lambda i: (i, 0, 0)),
        out_shape=jax.ShapeDtypeStruct((B, F, NUM_CLASSES), jnp.float32),
    )(x)


# speedup vs baseline: 1.1071x; 1.1071x over previous
import jax
import jax.numpy as jnp
from jax.experimental import pallas as pl

NUM_CLASSES = 1000
_BB = 64


def _body(x_ref, o_ref):
    xi = x_ref[...]
    iota = jax.lax.broadcasted_iota(jnp.int32, o_ref.shape, 2)
    o_ref[...] = (iota == xi).astype(jnp.float32)


def kernel(x):
    B, F, _ = x.shape
    return pl.pallas_call(
        _body,
        grid=(B // _BB,),
        in_specs=[pl.BlockSpec((_BB, F, 1), lambda i: (i, 0, 0))],
        out_specs=pl.BlockSpec((_BB, F, NUM_CLASSES), lambda i: (i, 0, 0)),
        out_shape=jax.ShapeDtypeStruct((B, F, NUM_CLASSES), jnp.float32),
    )(x)
